# Initial kernel scaffold; baseline (speedup 1.0000x reference)
import jax
import jax.numpy as jnp
from jax.experimental import pallas as pl


def kernel(x, batch, W):
    raise NotImplementedError("write your pallas kernel here")


# speedup vs baseline: 1.8250x; 1.8250x over previous
import jax
import jax.numpy as jnp
from jax import lax
from jax.experimental import pallas as pl
from jax.experimental.pallas import tpu as pltpu
from jax.experimental.pallas import tpu_sc as plsc

N = 100000
H = 128
S = 64
NW = 32
ROWS_PW = 3200
CHUNK = 160
NCH_FULL = ROWS_PW // CHUNK
NCH_LAST = (N - (NW - 1) * ROWS_PW) // CHUNK
NPAD = NW * ROWS_PW
NG = CHUNK // 16

_MESH = dict(core_axis_name="c", subcore_axis_name="s")


def _wid():
    return lax.axis_index("c") * 16 + lax.axis_index("s")


def _nchunks(wid):
    return jnp.where(wid == NW - 1, NCH_LAST, NCH_FULL)


def _pass1_body(x_hbm, b_hbm, w_hbm, gate_hbm, mpart_hbm,
                xbuf, bbuf, gstage, wv, mloc):
    wid = _wid()
    base = wid * ROWS_PW
    iota = lax.iota(jnp.int32, 16)
    pltpu.sync_copy(w_hbm, wv)
    wvec = [wv[pl.ds(16 * t, 16)] for t in range(H // 16)]
    for t in range(S // 16):
        mloc[t] = jnp.full((16,), -jnp.inf, jnp.float32)

    def chunk_body(ci, carry):
        r0 = base + ci * CHUNK
        pltpu.sync_copy(x_hbm.at[pl.ds(r0 * H, CHUNK * H)], xbuf)
        pltpu.sync_copy(b_hbm.at[pl.ds(r0, CHUNK)], bbuf)

        def group_body(gi, c2):
            rb = gi * 16
            idx0 = (rb + iota) * H
            b16 = bbuf[pl.ds(rb, 16)]
            acc = [jnp.zeros((16,), jnp.float32) for _ in range(8)]
            for j in range(H):
                col = plsc.load_gather(xbuf, [idx0 + j])
                acc[j % 8] = acc[j % 8] + col * wvec[j // 16][j % 16]
            g16 = ((acc[0] + acc[1]) + (acc[2] + acc[3])) + \
                  ((acc[4] + acc[5]) + (acc[6] + acc[7]))
            gstage[pl.ds(rb, 16)] = g16

            for r2 in range(16):
                g = g16[r2]
                b = b16[r2]
                t = b // 16
                lane = b - t * 16
                mv = mloc[t]
                mloc[t] = jnp.where(iota == lane,
                                    jnp.maximum(mv, g), mv)
            return c2

        lax.fori_loop(0, NG, group_body, 0)
        pltpu.sync_copy(gstage, gate_hbm.at[pl.ds(r0, CHUNK)])
        return carry

    lax.fori_loop(0, _nchunks(wid), chunk_body, 0)
    pltpu.sync_copy(mloc, mpart_hbm.at[wid])


def _pass2_body(x_hbm, b_hbm, gate_hbm, mpart_hbm, spart_hbm, accpart_hbm,
                xbuf, xw, bidx, bbuf, gcb, wbuf, mp, gmax, sloc, accsh):
    cid = lax.axis_index("c")
    sid = lax.axis_index("s")
    wid = cid * 16 + sid
    base = wid * ROWS_PW
    iota = lax.iota(jnp.int32, 16)

    pltpu.sync_copy(mpart_hbm, mp)
    for t in range(S // 16):
        mx = jnp.full((16,), -jnp.inf, jnp.float32)
        for w2 in range(NW):
            mx = jnp.maximum(mx, mp[w2, t])
        gmax[pl.ds(16 * t, 16)] = mx
    for t in range(S // 16):
        sloc[t] = jnp.zeros((16,), jnp.float32)

    @pl.when(sid == 0)
    def _():
        def zrow(r, c2):
            for j in range(H // 16):
                xw[r, pl.ds(16 * j, 16)] = jnp.zeros((16,), jnp.float32)
            return c2
        lax.fori_loop(0, S, zrow, 0)
        pltpu.sync_copy(xw.at[pl.ds(0, S), :], accsh)

    plsc.subcore_barrier()

    def chunk_body(ci, carry):
        r0 = base + ci * CHUNK
        pltpu.sync_copy(x_hbm.at[pl.ds(r0, CHUNK), :], xbuf)
        pltpu.sync_copy(b_hbm.at[pl.ds(r0, CHUNK // 2)], bidx.at[0])
        pltpu.sync_copy(b_hbm.at[pl.ds(r0 + CHUNK // 2, CHUNK // 2)],
                        bidx.at[1])
        pltpu.sync_copy(b_hbm.at[pl.ds(r0, CHUNK)], bbuf)
        pltpu.sync_copy(gate_hbm.at[pl.ds(r0, CHUNK)], gcb)

        for t in range(NG):
            b16 = bbuf[pl.ds(16 * t, 16)]
            m16 = plsc.load_gather(gmax, [b16])
            wbuf[pl.ds(16 * t, 16)] = jnp.exp(gcb[pl.ds(16 * t, 16)] - m16)

        def group_body(gi, c2):
            rb = gi * 16
            w16 = wbuf[pl.ds(rb, 16)]
            b16 = bbuf[pl.ds(rb, 16)]
            for r2 in range(16):
                r = rb + r2
                w = w16[r2]
                for j in range(H // 16):
                    sl = pl.ds(16 * j, 16)
                    xw[r, sl] = xbuf[r, sl] * w
                b = b16[r2]
                t = b // 16
                lane = b - t * 16
                sv = sloc[t]
                sloc[t] = jnp.where(iota == lane, sv + w, sv)
            return c2

        lax.fori_loop(0, NG, group_body, 0)

        pltpu.sync_copy(xw.at[pl.ds(0, CHUNK // 2), :],
                        accsh.at[bidx.at[0]], add=True)
        pltpu.sync_copy(xw.at[pl.ds(CHUNK // 2, CHUNK // 2), :],
                        accsh.at[bidx.at[1]], add=True)
        return carry

    lax.fori_loop(0, _nchunks(wid), chunk_body, 0)
    pltpu.sync_copy(sloc, spart_hbm.at[wid])
    plsc.subcore_barrier()

    @pl.when(sid == 0)
    def _():
        pltpu.sync_copy(accsh, accpart_hbm.at[cid])


def _merge_body(spart_hbm, accpart_hbm, out_hbm, sp, svv, a0, a1, ob):
    wid = _wid()
    iota = lax.iota(jnp.int32, 16)
    pltpu.sync_copy(spart_hbm, sp)
    for t in range(S // 16):
        sv = sp[0, t]
        for w2 in range(1, NW):
            sv = sv + sp[w2, t]
        svv[pl.ds(16 * t, 16)] = sv
    for k in range(S // NW):
        seg = wid * (S // NW) + k
        pltpu.sync_copy(accpart_hbm.at[0, seg], a0)
        pltpu.sync_copy(accpart_hbm.at[1, seg], a1)
        sg = plsc.load_gather(svv, [iota * 0 + seg])
        inv = 1.0 / (sg + 1e-16)
        for j in range(H // 16):
            sl = pl.ds(16 * j, 16)
            ob[sl] = (a0[sl] + a1[sl]) * inv
        pltpu.sync_copy(ob, out_hbm.at[seg])


def _make_kernels():
    mesh = plsc.VectorSubcoreMesh(**_MESH)
    cp = pltpu.CompilerParams(needs_layout_passes=False)
    p1 = pl.kernel(
        _pass1_body,
        out_type=[jax.ShapeDtypeStruct((NPAD,), jnp.float32),
                  jax.ShapeDtypeStruct((NW, S // 16, 16), jnp.float32)],
        mesh=mesh,
        scratch_types=[pltpu.VMEM((CHUNK * H,), jnp.float32),
                       pltpu.VMEM((CHUNK,), jnp.int32),
                       pltpu.VMEM((CHUNK,), jnp.float32),
                       pltpu.VMEM((H,), jnp.float32),
                       pltpu.VMEM((S // 16, 16), jnp.float32)],
        compiler_params=cp,
    )
    p2 = pl.kernel(
        _pass2_body,
        out_type=[jax.ShapeDtypeStruct((NW, S // 16, 16), jnp.float32),
                  jax.ShapeDtypeStruct((2, S, H), jnp.float32)],
        mesh=mesh,
        scratch_types=[pltpu.VMEM((CHUNK, H), jnp.float32),
                       pltpu.VMEM((CHUNK, H), jnp.float32),
                       pltpu.VMEM((2, CHUNK // 2), jnp.int32),
                       pltpu.VMEM((CHUNK,), jnp.int32),
                       pltpu.VMEM((CHUNK,), jnp.float32),
                       pltpu.VMEM((CHUNK,), jnp.float32),
                       pltpu.VMEM((NW, S // 16, 16), jnp.float32),
                       pltpu.VMEM((S,), jnp.float32),
                       pltpu.VMEM((S // 16, 16), jnp.float32),
                       pltpu.VMEM_SHARED((S, H), jnp.float32)],
        compiler_params=cp,
    )
    pm = pl.kernel(
        _merge_body,
        out_type=jax.ShapeDtypeStruct((S, H), jnp.float32),
        mesh=mesh,
        scratch_types=[pltpu.VMEM((NW, S // 16, 16), jnp.float32),
                       pltpu.VMEM((S,), jnp.float32),
                       pltpu.VMEM((H,), jnp.float32),
                       pltpu.VMEM((H,), jnp.float32),
                       pltpu.VMEM((H,), jnp.float32)],
        compiler_params=cp,
    )
    return p1, p2, pm


_P1, _P2, _PM = _make_kernels()


@jax.jit
def kernel(x, batch, W):
    w = W.reshape((H,))
    gate, mpart = _P1(x.reshape((N * H,)), batch, w)
    spart, accpart = _P2(x, batch, gate, mpart)
    return _PM(spart, accpart)


# speedup vs baseline: 2.6177x; 1.4344x over previous
import jax
import jax.numpy as jnp
from jax import lax
from jax.experimental import pallas as pl
from jax.experimental.pallas import tpu as pltpu
from jax.experimental.pallas import tpu_sc as plsc

N = 100000
H = 128
S = 64
NW = 32
ROWS_PW = 3200
CHUNK = 80
NCH_FULL = ROWS_PW // CHUNK
NCH_LAST = (N - (NW - 1) * ROWS_PW) // CHUNK
NPAD = NW * ROWS_PW
NGC = CHUNK // 16

_MESH = dict(core_axis_name="c", subcore_axis_name="s")


def _wid():
    return lax.axis_index("c") * 16 + lax.axis_index("s")


def _nchunks(wid):
    return jnp.where(wid == NW - 1, NCH_LAST, NCH_FULL)


def _xcopy(x_hbm, base, ci, buf, sem):
    src = x_hbm.at[pl.ds((base + ci * CHUNK) * H, CHUNK * H)]
    return pltpu.make_async_copy(src, buf, sem)


def _pass1_body(x_hbm, b_hbm, w_hbm, gate_hbm, mpart_hbm,
                xb0, xb1, bbuf, gbuf, wv, mloc, sem):
    wid = _wid()
    base = wid * ROWS_PW
    nch = _nchunks(wid)
    iota = lax.iota(jnp.int32, 16)
    pltpu.sync_copy(w_hbm, wv)
    pltpu.sync_copy(b_hbm.at[pl.ds(base, ROWS_PW)], bbuf)
    wvec = [wv[pl.ds(16 * t, 16)] for t in range(H // 16)]
    ws = [wvec[j // 16][j % 16] for j in range(H)]
    for t in range(S // 16):
        mloc[t] = jnp.full((16,), -jnp.inf, jnp.float32)

    def compute(ci, xb):
        def group_body(gi, c2):
            rb = gi * 16
            idx0 = (rb + iota) * H
            b16 = bbuf[pl.ds(ci * CHUNK + rb, 16)]
            acc = [jnp.zeros((16,), jnp.float32) for _ in range(8)]
            for j in range(H):
                col = plsc.load_gather(xb, [idx0 + j])
                acc[j % 8] = acc[j % 8] + col * ws[j]
            g16 = ((acc[0] + acc[1]) + (acc[2] + acc[3])) + \
                  ((acc[4] + acc[5]) + (acc[6] + acc[7]))
            gbuf[pl.ds(ci * CHUNK + rb, 16)] = g16

            def fast():
                g = jnp.max(g16)
                b = b16[0]
                t = b // 16
                lane = b - t * 16
                mv = mloc[t]
                mloc[t] = jnp.where(iota == lane,
                                    jnp.maximum(mv, g), mv)

            def slow():
                for r2 in range(16):
                    g = g16[r2]
                    b = b16[r2]
                    t = b // 16
                    lane = b - t * 16
                    mv = mloc[t]
                    mloc[t] = jnp.where(iota == lane,
                                        jnp.maximum(mv, g), mv)

            lax.cond(b16[0] == b16[15], fast, slow)
            return c2

        lax.fori_loop(0, NGC, group_body, 0)

    _xcopy(x_hbm, base, 0, xb0, sem).start()

    def pair_body(p, carry):
        ci0 = 2 * p
        ci1 = 2 * p + 1
        _xcopy(x_hbm, base, ci0, xb0, sem).wait()
        _xcopy(x_hbm, base, ci1, xb1, sem).start()
        compute(ci0, xb0)
        _xcopy(x_hbm, base, ci1, xb1, sem).wait()

        @pl.when(ci1 + 1 < nch)
        def _():
            _xcopy(x_hbm, base, ci1 + 1, xb0, sem).start()

        compute(ci1, xb1)
        return carry

    lax.fori_loop(0, nch // 2, pair_body, 0)
    pltpu.sync_copy(gbuf, gate_hbm.at[pl.ds(base, ROWS_PW)])
    pltpu.sync_copy(mloc, mpart_hbm.at[wid])


def _pass2_body(x_hbm, b2_hbm, gate_hbm, mpart_hbm, spart_hbm, accpart_hbm,
                xb0, xb1, xw, bidx, gcb, mp, gmax, sloc, accsh, sem):
    cid = lax.axis_index("c")
    sid = lax.axis_index("s")
    wid = cid * 16 + sid
    base = wid * ROWS_PW
    nch = _nchunks(wid)
    iota = lax.iota(jnp.int32, 16)

    pltpu.sync_copy(mpart_hbm, mp)
    for t in range(S // 16):
        mx = jnp.full((16,), -jnp.inf, jnp.float32)
        for w2 in range(NW):
            mx = jnp.maximum(mx, mp[w2, t])
        gmax[pl.ds(16 * t, 16)] = mx
    for t in range(S // 16):
        sloc[t] = jnp.zeros((16,), jnp.float32)
    pltpu.sync_copy(b2_hbm.at[pl.ds(wid * NCH_FULL, NCH_FULL), :], bidx)
    pltpu.sync_copy(gate_hbm.at[pl.ds(base, ROWS_PW)], gcb)

    @pl.when(sid == 0)
    def _():
        def zrow(r, c2):
            for j in range(H // 16):
                xw[r, pl.ds(16 * j, 16)] = jnp.zeros((16,), jnp.float32)
            return c2
        lax.fori_loop(0, S, zrow, 0)
        pltpu.sync_copy(xw.at[pl.ds(0, S), :], accsh)

    plsc.subcore_barrier()

    def compute(ci, xb):
        def group_body(gi, c2):
            rb = gi * 16
            b16 = bidx[ci, pl.ds(rb, 16)]
            m16 = plsc.load_gather(gmax, [b16])
            g16 = gcb[pl.ds(ci * CHUNK + rb, 16)]
            w16 = jnp.exp(g16 - m16)
            for r2 in range(16):
                r = rb + r2
                w = w16[r2]
                for j in range(H // 16):
                    sl = pl.ds(16 * j, 16)
                    xw[r, sl] = xb[r, sl] * w

            def fast():
                sw = jnp.sum(w16)
                b = b16[0]
                t = b // 16
                lane = b - t * 16
                sv = sloc[t]
                sloc[t] = jnp.where(iota == lane, sv + sw, sv)

            def slow():
                for r2 in range(16):
                    b = b16[r2]
                    t = b // 16
                    lane = b - t * 16
                    sv = sloc[t]
                    sloc[t] = jnp.where(iota == lane, sv + w16[r2], sv)

            lax.cond(b16[0] == b16[15], fast, slow)
            return c2

        lax.fori_loop(0, NGC, group_body, 0)
        pltpu.sync_copy(xw, accsh.at[bidx.at[ci]], add=True)

    _xcopy2(x_hbm, base, 0, xb0, sem).start()

    def pair_body(p, carry):
        ci0 = 2 * p
        ci1 = 2 * p + 1
        _xcopy2(x_hbm, base, ci0, xb0, sem).wait()
        _xcopy2(x_hbm, base, ci1, xb1, sem).start()
        compute(ci0, xb0)
        _xcopy2(x_hbm, base, ci1, xb1, sem).wait()

        @pl.when(ci1 + 1 < nch)
        def _():
            _xcopy2(x_hbm, base, ci1 + 1, xb0, sem).start()

        compute(ci1, xb1)
        return carry

    lax.fori_loop(0, nch // 2, pair_body, 0)
    pltpu.sync_copy(sloc, spart_hbm.at[wid])
    plsc.subcore_barrier()

    @pl.when(sid == 0)
    def _():
        pltpu.sync_copy(accsh, accpart_hbm.at[cid])


def _xcopy2(x_hbm, base, ci, buf, sem):
    src = x_hbm.at[pl.ds(base + ci * CHUNK, CHUNK), :]
    return pltpu.make_async_copy(src, buf, sem)


def _merge_body(spart_hbm, accpart_hbm, out_hbm, sp, svv, a0, a1, ob):
    wid = _wid()
    iota = lax.iota(jnp.int32, 16)
    pltpu.sync_copy(spart_hbm, sp)
    for t in range(S // 16):
        sv = sp[0, t]
        for w2 in range(1, NW):
            sv = sv + sp[w2, t]
        svv[pl.ds(16 * t, 16)] = sv
    for k in range(S // NW):
        seg = wid * (S // NW) + k
        pltpu.sync_copy(accpart_hbm.at[0, seg], a0)
        pltpu.sync_copy(accpart_hbm.at[1, seg], a1)
        sg = plsc.load_gather(svv, [iota * 0 + seg])
        inv = 1.0 / (sg + 1e-16)
        for j in range(H // 16):
            sl = pl.ds(16 * j, 16)
            ob[sl] = (a0[sl] + a1[sl]) * inv
        pltpu.sync_copy(ob, out_hbm.at[seg])


def _make_kernels():
    mesh = plsc.VectorSubcoreMesh(**_MESH)
    cp = pltpu.CompilerParams(needs_layout_passes=False)
    p1 = pl.kernel(
        _pass1_body,
        out_type=[jax.ShapeDtypeStruct((NPAD,), jnp.float32),
                  jax.ShapeDtypeStruct((NW, S // 16, 16), jnp.float32)],
        mesh=mesh,
        scratch_types=[pltpu.VMEM((CHUNK * H,), jnp.float32),
                       pltpu.VMEM((CHUNK * H,), jnp.float32),
                       pltpu.VMEM((ROWS_PW,), jnp.int32),
                       pltpu.VMEM((ROWS_PW,), jnp.float32),
                       pltpu.VMEM((H,), jnp.float32),
                       pltpu.VMEM((S // 16, 16), jnp.float32),
                       pltpu.SemaphoreType.DMA],
        compiler_params=cp,
    )
    p2 = pl.kernel(
        _pass2_body,
        out_type=[jax.ShapeDtypeStruct((NW, S // 16, 16), jnp.float32),
                  jax.ShapeDtypeStruct((2, S, H), jnp.float32)],
        mesh=mesh,
        scratch_types=[pltpu.VMEM((CHUNK, H), jnp.float32),
                       pltpu.VMEM((CHUNK, H), jnp.float32),
                       pltpu.VMEM((CHUNK, H), jnp.float32),
                       pltpu.VMEM((NCH_FULL, CHUNK), jnp.int32),
                       pltpu.VMEM((ROWS_PW,), jnp.float32),
                       pltpu.VMEM((NW, S // 16, 16), jnp.float32),
                       pltpu.VMEM((S,), jnp.float32),
                       pltpu.VMEM((S // 16, 16), jnp.float32),
                       pltpu.VMEM_SHARED((S, H), jnp.float32),
                       pltpu.SemaphoreType.DMA],
        compiler_params=cp,
    )
    pm = pl.kernel(
        _merge_body,
        out_type=jax.ShapeDtypeStruct((S, H), jnp.float32),
        mesh=mesh,
        scratch_types=[pltpu.VMEM((NW, S // 16, 16), jnp.float32),
                       pltpu.VMEM((S,), jnp.float32),
                       pltpu.VMEM((H,), jnp.float32),
                       pltpu.VMEM((H,), jnp.float32),
                       pltpu.VMEM((H,), jnp.float32)],
        compiler_params=cp,
    )
    return p1, p2, pm


_P1, _P2, _PM = _make_kernels()


@jax.jit
def kernel(x, batch, W):
    w = W.reshape((H,))
    batch_p = jnp.pad(batch, (0, NPAD - N), constant_values=S - 1)
    gate, mpart = _P1(x.reshape((N * H,)), batch_p, w)
    b2 = batch_p.reshape((NPAD // CHUNK, CHUNK))
    spart, accpart = _P2(x, b2, gate, mpart)
    return _PM(spart, accpart)


# speedup vs baseline: 5.5824x; 2.1325x over previous
import jax
import jax.numpy as jnp
from jax import lax
from jax.experimental import pallas as pl
from jax.experimental.pallas import tpu as pltpu
from jax.experimental.pallas import tpu_sc as plsc

N = 100000
H = 128
S = 64
NW = 32
ROWS_PW = 3200
CHUNK = 80
NCH_FULL = ROWS_PW // CHUNK
NCH_LAST = (N - (NW - 1) * ROWS_PW) // CHUNK
NPAD = NW * ROWS_PW
NGC = CHUNK // 16

_MESH = dict(core_axis_name="c", subcore_axis_name="s")


def _wid():
    return lax.axis_index("c") * 16 + lax.axis_index("s")


def _nchunks(wid):
    return jnp.where(wid == NW - 1, NCH_LAST, NCH_FULL)


def _xcopy2(x_hbm, base, ci, buf, sem):
    src = x_hbm.at[pl.ds(base + ci * CHUNK, CHUNK), :]
    return pltpu.make_async_copy(src, buf, sem)


def _pass1_body(x_hbm, b_hbm, w_hbm, gate_hbm, mpart_hbm,
                xb0, xb1, bbuf, gbuf, wv, mloc, sem):
    wid = _wid()
    base = wid * ROWS_PW
    nch = _nchunks(wid)
    iota = lax.iota(jnp.int32, 16)
    pltpu.sync_copy(w_hbm, wv)
    pltpu.sync_copy(b_hbm.at[pl.ds(base, ROWS_PW)], bbuf)
    wvec = [wv[pl.ds(16 * t, 16)] for t in range(H // 16)]
    for t in range(S // 16):
        mloc[t] = jnp.full((16,), -jnp.inf, jnp.float32)

    def compute(ci, xb):
        def group_body(gi, c2):
            rb = gi * 16
            b16 = bbuf[pl.ds(ci * CHUNK + rb, 16)]
            g16 = jnp.zeros((16,), jnp.float32)
            for r2 in range(16):
                r = rb + r2
                a = xb[r, pl.ds(0, 16)] * wvec[0]
                for j in range(1, H // 16):
                    a = a + xb[r, pl.ds(16 * j, 16)] * wvec[j]
                g16 = jnp.where(iota == r2, jnp.sum(a), g16)
            gbuf[pl.ds(ci * CHUNK + rb, 16)] = g16

            def fast():
                g = jnp.max(g16)
                b = b16[0]
                t = b // 16
                lane = b - t * 16
                mv = mloc[t]
                mloc[t] = jnp.where(iota == lane,
                                    jnp.maximum(mv, g), mv)

            def slow():
                for r2 in range(16):
                    g = g16[r2]
                    b = b16[r2]
                    t = b // 16
                    lane = b - t * 16
                    mv = mloc[t]
                    mloc[t] = jnp.where(iota == lane,
                                        jnp.maximum(mv, g), mv)

            lax.cond(b16[0] == b16[15], fast, slow)
            return c2

        lax.fori_loop(0, NGC, group_body, 0)

    _xcopy2(x_hbm, base, 0, xb0, sem).start()

    def pair_body(p, carry):
        ci0 = 2 * p
        ci1 = 2 * p + 1
        _xcopy2(x_hbm, base, ci0, xb0, sem).wait()
        _xcopy2(x_hbm, base, ci1, xb1, sem).start()
        compute(ci0, xb0)
        _xcopy2(x_hbm, base, ci1, xb1, sem).wait()

        @pl.when(ci1 + 1 < nch)
        def _():
            _xcopy2(x_hbm, base, ci1 + 1, xb0, sem).start()

        compute(ci1, xb1)
        return carry

    lax.fori_loop(0, nch // 2, pair_body, 0)
    pltpu.sync_copy(gbuf, gate_hbm.at[pl.ds(base, ROWS_PW)])
    pltpu.sync_copy(mloc, mpart_hbm.at[wid])


def _pass2_body(x_hbm, b2_hbm, gate_hbm, mpart_hbm, spart_hbm, accpart_hbm,
                xb0, xb1, xw0, xw1, bidx, gcb, mp, gmax, sloc, accsh,
                sem, sem_s):
    cid = lax.axis_index("c")
    sid = lax.axis_index("s")
    wid = cid * 16 + sid
    base = wid * ROWS_PW
    nch = _nchunks(wid)
    iota = lax.iota(jnp.int32, 16)

    pltpu.sync_copy(mpart_hbm, mp)
    for t in range(S // 16):
        mx = jnp.full((16,), -jnp.inf, jnp.float32)
        for w2 in range(NW):
            mx = jnp.maximum(mx, mp[w2, t])
        gmax[pl.ds(16 * t, 16)] = mx
    for t in range(S // 16):
        sloc[t] = jnp.zeros((16,), jnp.float32)
    pltpu.sync_copy(b2_hbm.at[pl.ds(wid * NCH_FULL, NCH_FULL), :], bidx)
    pltpu.sync_copy(gate_hbm.at[pl.ds(base, ROWS_PW)], gcb)

    @pl.when(sid == 0)
    def _():
        def zrow(r, c2):
            for j in range(H // 16):
                xw0[r, pl.ds(16 * j, 16)] = jnp.zeros((16,), jnp.float32)
            return c2
        lax.fori_loop(0, S, zrow, 0)
        pltpu.sync_copy(xw0.at[pl.ds(0, S), :], accsh)

    plsc.subcore_barrier()

    def compute(ci, xb, xwb):
        def group_body(gi, c2):
            rb = gi * 16
            b16 = bidx[ci, pl.ds(rb, 16)]
            m16 = plsc.load_gather(gmax, [b16])
            g16 = gcb[pl.ds(ci * CHUNK + rb, 16)]
            w16 = jnp.exp(g16 - m16)
            for r2 in range(16):
                r = rb + r2
                w = w16[r2]
                for j in range(H // 16):
                    sl = pl.ds(16 * j, 16)
                    xwb[r, sl] = xb[r, sl] * w

            def fast():
                sw = jnp.sum(w16)
                b = b16[0]
                t = b // 16
                lane = b - t * 16
                sv = sloc[t]
                sloc[t] = jnp.where(iota == lane, sv + sw, sv)

            def slow():
                for r2 in range(16):
                    b = b16[r2]
                    t = b // 16
                    lane = b - t * 16
                    sv = sloc[t]
                    sloc[t] = jnp.where(iota == lane, sv + w16[r2], sv)

            lax.cond(b16[0] == b16[15], fast, slow)
            return c2

        lax.fori_loop(0, NGC, group_body, 0)

    _xcopy2(x_hbm, base, 0, xb0, sem).start()

    def pair_body(p, carry):
        ci0 = 2 * p
        ci1 = 2 * p + 1
        _xcopy2(x_hbm, base, ci0, xb0, sem).wait()
        _xcopy2(x_hbm, base, ci1, xb1, sem).start()

        @pl.when(p > 0)
        def _():
            pltpu.make_async_copy(xw0, accsh.at[bidx.at[ci0]],
                                  sem_s).wait()

        compute(ci0, xb0, xw0)
        pltpu.async_copy(xw0, accsh.at[bidx.at[ci0]], sem_s, add=True)
        _xcopy2(x_hbm, base, ci1, xb1, sem).wait()

        @pl.when(ci1 + 1 < nch)
        def _():
            _xcopy2(x_hbm, base, ci1 + 1, xb0, sem).start()

        @pl.when(p > 0)
        def _():
            pltpu.make_async_copy(xw1, accsh.at[bidx.at[ci1]],
                                  sem_s).wait()

        compute(ci1, xb1, xw1)
        pltpu.async_copy(xw1, accsh.at[bidx.at[ci1]], sem_s, add=True)
        return carry

    lax.fori_loop(0, nch // 2, pair_body, 0)
    pltpu.make_async_copy(xw0, accsh.at[bidx.at[nch - 2]], sem_s).wait()
    pltpu.make_async_copy(xw1, accsh.at[bidx.at[nch - 1]], sem_s).wait()
    pltpu.sync_copy(sloc, spart_hbm.at[wid])
    plsc.subcore_barrier()

    @pl.when(sid == 0)
    def _():
        pltpu.sync_copy(accsh, accpart_hbm.at[cid])


def _merge_body(spart_hbm, accpart_hbm, out_hbm, sp, svv, a0, a1, ob):
    wid = _wid()
    iota = lax.iota(jnp.int32, 16)
    pltpu.sync_copy(spart_hbm, sp)
    for t in range(S // 16):
        sv = sp[0, t]
        for w2 in range(1, NW):
            sv = sv + sp[w2, t]
        svv[pl.ds(16 * t, 16)] = sv
    for k in range(S // NW):
        seg = wid * (S // NW) + k
        pltpu.sync_copy(accpart_hbm.at[0, seg], a0)
        pltpu.sync_copy(accpart_hbm.at[1, seg], a1)
        sg = plsc.load_gather(svv, [iota * 0 + seg])
        inv = 1.0 / (sg + 1e-16)
        for j in range(H // 16):
            sl = pl.ds(16 * j, 16)
            ob[sl] = (a0[sl] + a1[sl]) * inv
        pltpu.sync_copy(ob, out_hbm.at[seg])


def _make_kernels():
    mesh = plsc.VectorSubcoreMesh(**_MESH)
    cp = pltpu.CompilerParams(needs_layout_passes=False)
    p1 = pl.kernel(
        _pass1_body,
        out_type=[jax.ShapeDtypeStruct((NPAD,), jnp.float32),
                  jax.ShapeDtypeStruct((NW, S // 16, 16), jnp.float32)],
        mesh=mesh,
        scratch_types=[pltpu.VMEM((CHUNK, H), jnp.float32),
                       pltpu.VMEM((CHUNK, H), jnp.float32),
                       pltpu.VMEM((ROWS_PW,), jnp.int32),
                       pltpu.VMEM((ROWS_PW,), jnp.float32),
                       pltpu.VMEM((H,), jnp.float32),
                       pltpu.VMEM((S // 16, 16), jnp.float32),
                       pltpu.SemaphoreType.DMA],
        compiler_params=cp,
    )
    p2 = pl.kernel(
        _pass2_body,
        out_type=[jax.ShapeDtypeStruct((NW, S // 16, 16), jnp.float32),
                  jax.ShapeDtypeStruct((2, S, H), jnp.float32)],
        mesh=mesh,
        scratch_types=[pltpu.VMEM((CHUNK, H), jnp.float32),
                       pltpu.VMEM((CHUNK, H), jnp.float32),
                       pltpu.VMEM((CHUNK, H), jnp.float32),
                       pltpu.VMEM((CHUNK, H), jnp.float32),
                       pltpu.VMEM((NCH_FULL, CHUNK), jnp.int32),
                       pltpu.VMEM((ROWS_PW,), jnp.float32),
                       pltpu.VMEM((NW, S // 16, 16), jnp.float32),
                       pltpu.VMEM((S,), jnp.float32),
                       pltpu.VMEM((S // 16, 16), jnp.float32),
                       pltpu.VMEM_SHARED((S, H), jnp.float32),
                       pltpu.SemaphoreType.DMA,
                       pltpu.SemaphoreType.DMA],
        compiler_params=cp,
    )
    pm = pl.kernel(
        _merge_body,
        out_type=jax.ShapeDtypeStruct((S, H), jnp.float32),
        mesh=mesh,
        scratch_types=[pltpu.VMEM((NW, S // 16, 16), jnp.float32),
                       pltpu.VMEM((S,), jnp.float32),
                       pltpu.VMEM((H,), jnp.float32),
                       pltpu.VMEM((H,), jnp.float32),
                       pltpu.VMEM((H,), jnp.float32)],
        compiler_params=cp,
    )
    return p1, p2, pm


_P1, _P2, _PM = _make_kernels()


@jax.jit
def kernel(x, batch, W):
    w = W.reshape((H,))
    batch_p = jnp.pad(batch, (0, NPAD - N), constant_values=S - 1)
    gate, mpart = _P1(x, batch_p, w)
    b2 = batch_p.reshape((NPAD // CHUNK, CHUNK))
    spart, accpart = _P2(x, b2, gate, mpart)
    return _PM(spart, accpart)


# speedup vs baseline: 8.7341x; 1.5646x over previous
import jax
import jax.numpy as jnp
from jax import lax
from jax.experimental import pallas as pl
from jax.experimental.pallas import tpu as pltpu
from jax.experimental.pallas import tpu_sc as plsc

N = 100000
H = 128
S = 64
NW = 32
ROWS_PW = 3200
CHUNK = 80
NCH_FULL = ROWS_PW // CHUNK
NCH_LAST = (N - (NW - 1) * ROWS_PW) // CHUNK
NPAD = NW * ROWS_PW
NGC = CHUNK // 16
NEG = float("-inf")

_MESH = dict(core_axis_name="c", subcore_axis_name="s")


def _wid():
    return lax.axis_index("c") * 16 + lax.axis_index("s")


def _nchunks(wid):
    return jnp.where(wid == NW - 1, NCH_LAST, NCH_FULL)


def _xcopy(x_hbm, base, ci, buf, sem):
    src = x_hbm.at[pl.ds(base + ci * CHUNK, CHUNK), :]
    return pltpu.make_async_copy(src, buf, sem)


def _main_body(x_hbm, b_hbm, w_hbm, mpart_hbm, spart_hbm, apart_hbm,
               xb0, xb1, bbuf, wv, mloc, sloc, abuf, sem):
    wid = _wid()
    base = wid * ROWS_PW
    nch = _nchunks(wid)
    iota = lax.iota(jnp.int32, 16)
    pltpu.sync_copy(w_hbm, wv)
    pltpu.sync_copy(b_hbm.at[pl.ds(base, ROWS_PW)], bbuf)
    _xcopy(x_hbm, base, 0, xb0, sem).start()
    wvec = [wv[pl.ds(16 * t, 16)] for t in range(H // 16)]
    for t in range(S // 16):
        mloc[pl.ds(16 * t, 16)] = jnp.full((16,), NEG, jnp.float32)
        sloc[pl.ds(16 * t, 16)] = jnp.zeros((16,), jnp.float32)

    def zrow(r, c2):
        for j in range(H // 16):
            abuf[r, pl.ds(16 * j, 16)] = jnp.zeros((16,), jnp.float32)
        return c2
    lax.fori_loop(0, S, zrow, 0)

    def compute(ci, xb):
        def group_body(gi, c2):
            rb = gi * 16
            b16 = bbuf[pl.ds(ci * CHUNK + rb, 16)]
            g16 = jnp.zeros((16,), jnp.float32)
            for r2 in range(16):
                a = xb[rb + r2, pl.ds(0, 16)] * wvec[0]
                for j in range(1, H // 16):
                    a = a + xb[rb + r2, pl.ds(16 * j, 16)] * wvec[j]
                g16 = jnp.where(iota == r2, jnp.sum(a), g16)

            def fast():
                b = b16[0]
                t = b // 16
                lane = b - t * 16
                gm = jnp.max(g16)
                mo = plsc.load_gather(mloc, [b16])
                mn = jnp.maximum(mo, gm)
                c = jnp.exp(mo - mn)
                w16 = jnp.exp(g16 - mn)
                mv = mloc[pl.ds(t * 16, 16)]
                mloc[pl.ds(t * 16, 16)] = jnp.where(iota == lane, mn, mv)
                sw = jnp.sum(w16)
                sv = sloc[pl.ds(t * 16, 16)]
                sloc[pl.ds(t * 16, 16)] = jnp.where(
                    iota == lane, sv * c + sw, sv)
                acc = [w16[0] * xb[rb, pl.ds(16 * j, 16)]
                       for j in range(H // 16)]
                for r2 in range(1, 16):
                    for j in range(H // 16):
                        acc[j] = acc[j] + \
                            w16[r2] * xb[rb + r2, pl.ds(16 * j, 16)]
                for j in range(H // 16):
                    sl = pl.ds(16 * j, 16)
                    abuf[b, sl] = abuf[b, sl] * c + acc[j]

            def slow():
                for r2 in range(16):
                    b = b16[r2]
                    t = b // 16
                    lane = b - t * 16
                    g = g16[r2]
                    mo = plsc.load_gather(mloc, [iota * 0 + b])
                    mn = jnp.maximum(mo, g)
                    c = jnp.exp(mo - mn)
                    w = jnp.exp(g - mn)
                    mv = mloc[pl.ds(t * 16, 16)]
                    mloc[pl.ds(t * 16, 16)] = jnp.where(
                        iota == lane, mn, mv)
                    sv = sloc[pl.ds(t * 16, 16)]
                    sloc[pl.ds(t * 16, 16)] = jnp.where(
                        iota == lane, sv * c + w, sv)
                    for j in range(H // 16):
                        sl = pl.ds(16 * j, 16)
                        abuf[b, sl] = abuf[b, sl] * c + \
                            w * xb[rb + r2, sl]

            lax.cond(b16[0] == b16[15], fast, slow)
            return c2

        lax.fori_loop(0, NGC, group_body, 0)

    def pair_body(p, carry):
        ci0 = 2 * p
        ci1 = 2 * p + 1
        _xcopy(x_hbm, base, ci0, xb0, sem).wait()
        _xcopy(x_hbm, base, ci1, xb1, sem).start()
        compute(ci0, xb0)
        _xcopy(x_hbm, base, ci1, xb1, sem).wait()

        @pl.when(ci1 + 1 < nch)
        def _():
            _xcopy(x_hbm, base, ci1 + 1, xb0, sem).start()

        compute(ci1, xb1)
        return carry

    lax.fori_loop(0, nch // 2, pair_body, 0)
    pltpu.sync_copy(mloc, mpart_hbm.at[wid])
    pltpu.sync_copy(sloc, spart_hbm.at[wid])
    pltpu.sync_copy(abuf, apart_hbm.at[wid])


def _merge_body(mpart_hbm, spart_hbm, a2_hbm, out_hbm,
                mp, sp, idxbuf, rows, ob, sem):
    wid = _wid()
    iota = lax.iota(jnp.int32, 16)
    pltpu.sync_copy(mpart_hbm, mp)
    pltpu.sync_copy(spart_hbm, sp)
    for k in range(S // NW):
        seg = wid * (S // NW) + k
        ia = iota * S + seg
        ib = ia + 16 * S
        m16a = plsc.load_gather(mp, [ia])
        m16b = plsc.load_gather(mp, [ib])
        mg = jnp.maximum(jnp.max(jnp.maximum(m16a, m16b)),
                         jnp.float32(-1e38))
        f16a = jnp.exp(m16a - mg)
        f16b = jnp.exp(m16b - mg)
        s16a = plsc.load_gather(sp, [ia])
        s16b = plsc.load_gather(sp, [ib])
        sden = jnp.sum(s16a * f16a) + jnp.sum(s16b * f16b)
        inv = 1.0 / (jnp.zeros((16,), jnp.float32) + sden + 1e-16)
        idxbuf[pl.ds(0, 16)] = ia
        idxbuf[pl.ds(16, 16)] = ib
        pltpu.async_copy(a2_hbm.at[idxbuf], rows, sem).wait()
        for j in range(H // 16):
            sl = pl.ds(16 * j, 16)
            o = f16a[0] * rows[0, sl]
            for w2 in range(1, NW):
                f = f16a[w2] if w2 < 16 else f16b[w2 - 16]
                o = o + f * rows[w2, sl]
            ob[sl] = o * inv
        pltpu.sync_copy(ob, out_hbm.at[seg])


def _make_kernels():
    mesh = plsc.VectorSubcoreMesh(**_MESH)
    cp = pltpu.CompilerParams(needs_layout_passes=False)
    pmain = pl.kernel(
        _main_body,
        out_type=[jax.ShapeDtypeStruct((NW, S), jnp.float32),
                  jax.ShapeDtypeStruct((NW, S), jnp.float32),
                  jax.ShapeDtypeStruct((NW, S, H), jnp.float32)],
        mesh=mesh,
        scratch_types=[pltpu.VMEM((CHUNK, H), jnp.float32),
                       pltpu.VMEM((CHUNK, H), jnp.float32),
                       pltpu.VMEM((ROWS_PW,), jnp.int32),
                       pltpu.VMEM((H,), jnp.float32),
                       pltpu.VMEM((S,), jnp.float32),
                       pltpu.VMEM((S,), jnp.float32),
                       pltpu.VMEM((S, H), jnp.float32),
                       pltpu.SemaphoreType.DMA],
        compiler_params=cp,
    )
    pmerge = pl.kernel(
        _merge_body,
        out_type=jax.ShapeDtypeStruct((S, H), jnp.float32),
        mesh=mesh,
        scratch_types=[pltpu.VMEM((NW * S,), jnp.float32),
                       pltpu.VMEM((NW * S,), jnp.float32),
                       pltpu.VMEM((NW,), jnp.int32),
                       pltpu.VMEM((NW, H), jnp.float32),
                       pltpu.VMEM((H,), jnp.float32),
                       pltpu.SemaphoreType.DMA],
        compiler_params=cp,
    )
    return pmain, pmerge


_PMAIN, _PMERGE = _make_kernels()


@jax.jit
def kernel(x, batch, W):
    w = W.reshape((H,))
    batch_p = jnp.pad(batch, (0, NPAD - N), constant_values=S - 1)
    mpart, spart, apart = _PMAIN(x, batch_p, w)
    return _PMERGE(mpart.reshape((NW * S,)), spart.reshape((NW * S,)),
                   apart.reshape((NW * S, H)))
